# baseline (speedup 1.0000x reference)
import functools

import jax
import jax.numpy as jnp
from jax import lax
from jax.experimental import pallas as pl
from jax.experimental.pallas import tpu as pltpu
from jax.experimental.pallas import tpu_sc as plsc

N = 10000
E = 320000
F = 128
G = 128
EPS_BN = 1e-4

BLK = 2000
NBLK = N // BLK

NC = 2
NS = 16
TILES = NC * NS
EPT = E // TILES
CH = 50
NCH = EPT // CH
RING = 4
RPT_A = N // NS
NPAD = 10240
RPT = NPAD // NS
RSTG = 128
DEGW = 16

_sc_mesh = plsc.VectorSubcoreMesh(
    core_axis_name="c", subcore_axis_name="s", num_cores=NC, num_subcores=NS)


def _deg_body(dst_hbm, ones_hbm, zeros_hbm, out_hbm, idx_d, ones_v, stage_v,
              shared, ssem0, ssem1):
    c = lax.axis_index("c")
    s = lax.axis_index("s")
    wid = c * NS + s
    ssem = (ssem0, ssem1)
    pltpu.sync_copy(zeros_hbm, stage_v)
    for k in range(RPT // RSTG):
        pltpu.sync_copy(stage_v, shared.at[pl.ds(s * RPT + k * RSTG, RSTG)])
    pltpu.sync_copy(ones_hbm, ones_v)
    pltpu.sync_copy(dst_hbm.at[wid], idx_d)
    plsc.subcore_barrier()

    def step(jo, carry):
        for bb in range(2):
            j = 2 * jo + bb

            @pl.when(j >= 2)
            def _():
                pltpu.make_async_copy(
                    ones_v, shared.at[idx_d.at[j - 2]], ssem[bb]).wait()

            @pl.when(j < NCH)
            def _():
                pltpu.async_copy(
                    ones_v, shared.at[idx_d.at[j]], ssem[bb], add=True)
        return carry

    lax.fori_loop(0, NCH // 2 + 1, step, 0)
    if NCH % 2:
        pltpu.make_async_copy(
            ones_v, shared.at[idx_d.at[NCH - 1]], ssem[0]).wait()
    plsc.subcore_barrier()
    for k in range(RPT // RSTG):
        pltpu.sync_copy(shared.at[pl.ds(s * RPT + k * RSTG, RSTG)], stage_v)
        pltpu.sync_copy(stage_v, out_hbm.at[c, pl.ds(s * RPT + k * RSTG, RSTG)])


_deg_call = functools.partial(
    pl.kernel,
    out_type=jax.ShapeDtypeStruct((NC, NPAD, DEGW), jnp.float32),
    mesh=_sc_mesh,
    scratch_types=[
        pltpu.VMEM((NCH, CH), jnp.int32),
        pltpu.VMEM((CH, DEGW), jnp.float32),
        pltpu.VMEM((RSTG, DEGW), jnp.float32),
        pltpu.VMEM_SHARED((NPAD, DEGW), jnp.float32),
        pltpu.SemaphoreType.DMA,
        pltpu.SemaphoreType.DMA,
    ],
    compiler_params=pltpu.CompilerParams(use_tc_tiling_on_sc=False),
)(_deg_body)


def _agg_body(hp_hbm, src_hbm, dst_hbm, zeros_hbm, out_hbm, idx_s, idx_d,
              rows0, rows1, rows2, rows3, gsem0, gsem1, gsem2, gsem3,
              ssem0, ssem1, ssem2, ssem3, shared):
    c = lax.axis_index("c")
    s = lax.axis_index("s")
    wid = c * NS + s
    rows = (rows0, rows1, rows2, rows3)
    gsem = (gsem0, gsem1, gsem2, gsem3)
    ssem = (ssem0, ssem1, ssem2, ssem3)
    tail = RPT_A % CH
    nfull = RPT_A // CH
    base = s * RPT_A
    pltpu.sync_copy(zeros_hbm, rows0)
    for k in range(nfull):
        pltpu.sync_copy(rows0, shared.at[pl.ds(base + k * CH, CH)])
    pltpu.sync_copy(rows0.at[pl.ds(0, tail)],
                    shared.at[pl.ds(base + nfull * CH, tail)])
    pltpu.sync_copy(src_hbm.at[wid], idx_s)
    pltpu.sync_copy(dst_hbm.at[wid], idx_d)
    plsc.subcore_barrier()

    def step(jo, carry):
        for bb in range(RING):
            j = RING * jo + bb
            bs = (bb + 2) % RING

            @pl.when((j >= RING) & (j < NCH + RING))
            def _():
                pltpu.make_async_copy(
                    rows[bb], shared.at[idx_d.at[j - RING]], ssem[bb]).wait()

            @pl.when(j < NCH)
            def _():
                pltpu.async_copy(hp_hbm.at[idx_s.at[j]], rows[bb], gsem[bb])

            @pl.when((j >= 2) & (j < NCH + 2))
            def _():
                pltpu.make_async_copy(
                    hp_hbm.at[idx_s.at[j - 2]], rows[bs], gsem[bs]).wait()
                pltpu.async_copy(
                    rows[bs], shared.at[idx_d.at[j - 2]], ssem[bs], add=True)
        return carry

    lax.fori_loop(0, (NCH + 2 * RING) // RING + 1, step, 0)
    plsc.subcore_barrier()
    for k in range(nfull):
        pltpu.sync_copy(shared.at[pl.ds(base + k * CH, CH)], rows0)
        pltpu.sync_copy(rows0, out_hbm.at[c, pl.ds(base + k * CH, CH)])
    pltpu.sync_copy(shared.at[pl.ds(base + nfull * CH, tail)],
                    rows0.at[pl.ds(0, tail)])
    pltpu.sync_copy(rows0.at[pl.ds(0, tail)],
                    out_hbm.at[c, pl.ds(base + nfull * CH, tail)])


_agg_call = functools.partial(
    pl.kernel,
    out_type=jax.ShapeDtypeStruct((NC, N, F), jnp.float32),
    mesh=_sc_mesh,
    scratch_types=[
        pltpu.VMEM((NCH, CH), jnp.int32),
        pltpu.VMEM((NCH, CH), jnp.int32),
        pltpu.VMEM((CH, F), jnp.float32),
        pltpu.VMEM((CH, F), jnp.float32),
        pltpu.VMEM((CH, F), jnp.float32),
        pltpu.VMEM((CH, F), jnp.float32),
        pltpu.SemaphoreType.DMA,
        pltpu.SemaphoreType.DMA,
        pltpu.SemaphoreType.DMA,
        pltpu.SemaphoreType.DMA,
        pltpu.SemaphoreType.DMA,
        pltpu.SemaphoreType.DMA,
        pltpu.SemaphoreType.DMA,
        pltpu.SemaphoreType.DMA,
        pltpu.VMEM_SHARED((N, F), jnp.float32),
    ],
    compiler_params=pltpu.CompilerParams(use_tc_tiling_on_sc=False),
)(_agg_body)


def _pre_body(x_ref, w_ref, deg_ref, hp_ref, dis_ref):
    d = deg_ref[0, :, 0:1] + deg_ref[1, :, 0:1] + 1.0
    dis = jnp.broadcast_to(lax.rsqrt(d), (BLK, F))
    h = jnp.dot(x_ref[...], w_ref[...], preferred_element_type=jnp.float32)
    hp_ref[...] = dis * h
    dis_ref[...] = dis


def _pre(x, w0, degp):
    return pl.pallas_call(
        _pre_body,
        grid=(NBLK,),
        in_specs=[
            pl.BlockSpec((BLK, F), lambda i: (i, 0)),
            pl.BlockSpec((F, F), lambda i: (0, 0)),
            pl.BlockSpec((NC, BLK, DEGW), lambda i: (0, i, 0)),
        ],
        out_specs=[
            pl.BlockSpec((BLK, F), lambda i: (i, 0)),
            pl.BlockSpec((BLK, F), lambda i: (i, 0)),
        ],
        out_shape=[
            jax.ShapeDtypeStruct((N, F), jnp.float32),
            jax.ShapeDtypeStruct((N, F), jnp.float32),
        ],
    )(x, w0, degp)


def _layer_body(agg_ref, hp_ref, dis_ref, b_ref, g_ref, be_ref, w_ref,
                o_ref, s_scr, st_scr):
    p = pl.program_id(0)
    i = pl.program_id(1)

    @pl.when(p == 0)
    def _():
        t = (dis_ref[...] * (agg_ref[0] + agg_ref[1] + hp_ref[...])
             + b_ref[...])
        sv = jax.nn.sigmoid(t)
        s_scr[pl.ds(i * BLK, BLK), :] = sv

        @pl.when(i == 0)
        def _():
            st_scr[...] = jnp.zeros_like(st_scr)

        st_scr[0:1, :] += jnp.sum(sv, axis=0, keepdims=True)
        st_scr[1:2, :] += jnp.sum(sv * sv, axis=0, keepdims=True)

    @pl.when(p == 1)
    def _():
        mean = st_scr[0:1, :] / N
        var = st_scr[1:2, :] / N - mean * mean
        sv = s_scr[pl.ds(i * BLK, BLK), :]
        y = (sv - mean) * lax.rsqrt(var + EPS_BN) * g_ref[...] + be_ref[...]
        o_ref[...] = dis_ref[...] * jnp.dot(
            y, w_ref[...], preferred_element_type=jnp.float32)


def _layer(aggp, hp, dis, b, g, be, wnext):
    return pl.pallas_call(
        _layer_body,
        grid=(2, NBLK),
        in_specs=[
            pl.BlockSpec((NC, BLK, F), lambda p, i: (0, i, 0)),
            pl.BlockSpec((BLK, F), lambda p, i: (i, 0)),
            pl.BlockSpec((BLK, F), lambda p, i: (i, 0)),
            pl.BlockSpec((1, F), lambda p, i: (0, 0)),
            pl.BlockSpec((1, F), lambda p, i: (0, 0)),
            pl.BlockSpec((1, F), lambda p, i: (0, 0)),
            pl.BlockSpec((F, F), lambda p, i: (0, 0)),
        ],
        out_specs=pl.BlockSpec((BLK, F), lambda p, i: (i, 0)),
        out_shape=jax.ShapeDtypeStruct((N, F), jnp.float32),
        scratch_shapes=[
            pltpu.VMEM((N, F), jnp.float32),
            pltpu.VMEM((8, F), jnp.float32),
        ],
    )(aggp, hp, dis, b, g, be, wnext)


def _last_body(agg_ref, hp_ref, dis_ref, b_ref, g_ref, be_ref, batch_ref,
               h_ref, xp_ref, s_scr, st_scr, ps_scr, pc_scr):
    p = pl.program_id(0)
    i = pl.program_id(1)

    @pl.when(p == 0)
    def _():
        t = (dis_ref[...] * (agg_ref[0] + agg_ref[1] + hp_ref[...])
             + b_ref[...])
        sv = jax.nn.sigmoid(t)
        s_scr[pl.ds(i * BLK, BLK), :] = sv

        @pl.when(i == 0)
        def _():
            st_scr[...] = jnp.zeros_like(st_scr)

        st_scr[0:1, :] += jnp.sum(sv, axis=0, keepdims=True)
        st_scr[1:2, :] += jnp.sum(sv * sv, axis=0, keepdims=True)

    @pl.when(p == 1)
    def _():
        mean = st_scr[0:1, :] / N
        var = st_scr[1:2, :] / N - mean * mean
        sv = s_scr[pl.ds(i * BLK, BLK), :]
        y = (sv - mean) * lax.rsqrt(var + EPS_BN) * g_ref[...] + be_ref[...]
        h_ref[...] = y

        oh = (batch_ref[...] == lax.broadcasted_iota(jnp.int32, (BLK, G), 1)
              ).astype(jnp.float32)

        @pl.when(i == 0)
        def _():
            ps_scr[...] = jnp.zeros_like(ps_scr)
            pc_scr[...] = jnp.zeros_like(pc_scr)

        ps_scr[...] += lax.dot_general(oh, y, (((0,), (0,)), ((), ())),
                                       preferred_element_type=jnp.float32)
        pc_scr[...] += jnp.sum(oh, axis=0, keepdims=True)

        @pl.when(i == NBLK - 1)
        def _():
            xp_ref[...] = (ps_scr[...]
                           / jnp.maximum(pc_scr[...], 1.0).reshape(G, 1))


def _last(aggp, hp, dis, b, g, be, batch2):
    return pl.pallas_call(
        _last_body,
        grid=(2, NBLK),
        in_specs=[
            pl.BlockSpec((NC, BLK, F), lambda p, i: (0, i, 0)),
            pl.BlockSpec((BLK, F), lambda p, i: (i, 0)),
            pl.BlockSpec((BLK, F), lambda p, i: (i, 0)),
            pl.BlockSpec((1, F), lambda p, i: (0, 0)),
            pl.BlockSpec((1, F), lambda p, i: (0, 0)),
            pl.BlockSpec((1, F), lambda p, i: (0, 0)),
            pl.BlockSpec((BLK, 1), lambda p, i: (i, 0)),
        ],
        out_specs=[
            pl.BlockSpec((BLK, F), lambda p, i: (i, 0)),
            pl.BlockSpec((G, G), lambda p, i: (0, 0)),
        ],
        out_shape=[
            jax.ShapeDtypeStruct((N, F), jnp.float32),
            jax.ShapeDtypeStruct((G, G), jnp.float32),
        ],
        scratch_shapes=[
            pltpu.VMEM((N, F), jnp.float32),
            pltpu.VMEM((8, F), jnp.float32),
            pltpu.VMEM((G, G), jnp.float32),
            pltpu.VMEM((1, G), jnp.float32),
        ],
    )(aggp, hp, dis, b, g, be, batch2)


def kernel(x, edge_index, batch, W0, b0, g0, be0, W1, b1, g1, be1,
           W2, b2, g2, be2):
    src = edge_index[0].reshape(TILES, NCH, CH)
    dst = edge_index[1].reshape(TILES, NCH, CH)
    batch2 = batch.reshape(N, 1)
    ones_deg = jnp.ones((CH, DEGW), jnp.float32)
    zeros_deg = jnp.zeros((RSTG, DEGW), jnp.float32)
    zeros_f = jnp.zeros((CH, F), jnp.float32)

    degp = _deg_call(dst, ones_deg, zeros_deg)
    hp, dis = _pre(x, W0, degp)

    params = [(b0, g0, be0, W1), (b1, g1, be1, W2), (b2, g2, be2, None)]
    for (b, g, be, wnext) in params:
        aggp = _agg_call(hp, src, dst, zeros_f)
        if wnext is not None:
            hp = _layer(aggp, hp, dis, b.reshape(1, F), g.reshape(1, F),
                        be.reshape(1, F), wnext)
        else:
            h, xpool = _last(aggp, hp, dis, b.reshape(1, F), g.reshape(1, F),
                             be.reshape(1, F), batch2)
    return (xpool, h)


# speedup vs baseline: 1.0867x; 1.0867x over previous
import functools

import jax
import jax.numpy as jnp
from jax import lax
from jax.experimental import pallas as pl
from jax.experimental.pallas import tpu as pltpu
from jax.experimental.pallas import tpu_sc as plsc

N = 10000
E = 320000
F = 128
G = 128
EPS_BN = 1e-4

BLK = 2000
NBLK = N // BLK

NC = 2
NS = 16
TILES = NC * NS
EPT = E // TILES
CH = 80
NCH = EPT // CH
RING = 3
RPT_A = N // NS
NPAD = 10240
RPT = NPAD // NS
RSTG = 128
DEGW = 16

_sc_mesh = plsc.VectorSubcoreMesh(
    core_axis_name="c", subcore_axis_name="s", num_cores=NC, num_subcores=NS)


def _deg_body(dst_hbm, ones_hbm, zeros_hbm, out_hbm, idx_d, ones_v, stage_v,
              shared, ssem0, ssem1):
    c = lax.axis_index("c")
    s = lax.axis_index("s")
    wid = c * NS + s
    ssem = (ssem0, ssem1)
    pltpu.sync_copy(zeros_hbm, stage_v)
    for k in range(RPT // RSTG):
        pltpu.sync_copy(stage_v, shared.at[pl.ds(s * RPT + k * RSTG, RSTG)])
    pltpu.sync_copy(ones_hbm, ones_v)
    pltpu.sync_copy(dst_hbm.at[wid], idx_d)
    plsc.subcore_barrier()

    def step(jo, carry):
        for bb in range(2):
            j = 2 * jo + bb

            @pl.when(j >= 2)
            def _():
                pltpu.make_async_copy(
                    ones_v, shared.at[idx_d.at[j - 2]], ssem[bb]).wait()

            @pl.when(j < NCH)
            def _():
                pltpu.async_copy(
                    ones_v, shared.at[idx_d.at[j]], ssem[bb], add=True)
        return carry

    lax.fori_loop(0, NCH // 2 + 1, step, 0)
    if NCH % 2:
        pltpu.make_async_copy(
            ones_v, shared.at[idx_d.at[NCH - 1]], ssem[0]).wait()
    plsc.subcore_barrier()
    for k in range(RPT // RSTG):
        pltpu.sync_copy(shared.at[pl.ds(s * RPT + k * RSTG, RSTG)], stage_v)
        pltpu.sync_copy(stage_v, out_hbm.at[c, pl.ds(s * RPT + k * RSTG, RSTG)])


_deg_call = functools.partial(
    pl.kernel,
    out_type=jax.ShapeDtypeStruct((NC, NPAD, DEGW), jnp.float32),
    mesh=_sc_mesh,
    scratch_types=[
        pltpu.VMEM((NCH, CH), jnp.int32),
        pltpu.VMEM((CH, DEGW), jnp.float32),
        pltpu.VMEM((RSTG, DEGW), jnp.float32),
        pltpu.VMEM_SHARED((NPAD, DEGW), jnp.float32),
        pltpu.SemaphoreType.DMA,
        pltpu.SemaphoreType.DMA,
    ],
    compiler_params=pltpu.CompilerParams(use_tc_tiling_on_sc=False),
)(_deg_body)


def _agg_body(hp_hbm, src_hbm, dst_hbm, zeros_hbm, out_hbm, idx_s, idx_d,
              rows0, rows1, rows2, gsem0, gsem1, gsem2,
              ssem0, ssem1, ssem2, shared):
    c = lax.axis_index("c")
    s = lax.axis_index("s")
    wid = c * NS + s
    rows = (rows0, rows1, rows2)
    gsem = (gsem0, gsem1, gsem2)
    ssem = (ssem0, ssem1, ssem2)
    tail = RPT_A % CH
    nfull = RPT_A // CH
    base = s * RPT_A
    pltpu.sync_copy(zeros_hbm, rows0)
    for k in range(nfull):
        pltpu.sync_copy(rows0, shared.at[pl.ds(base + k * CH, CH)])
    pltpu.sync_copy(rows0.at[pl.ds(0, tail)],
                    shared.at[pl.ds(base + nfull * CH, tail)])
    pltpu.sync_copy(src_hbm.at[pl.ds(wid * EPT, EPT)], idx_s)
    pltpu.sync_copy(dst_hbm.at[wid], idx_d)
    plsc.subcore_barrier()

    def step(jo, carry):
        for bb in range(RING):
            j = RING * jo + bb
            bs = (bb + 1) % RING

            @pl.when((j >= RING) & (j < NCH + RING))
            def _():
                pltpu.make_async_copy(
                    rows[bb], shared.at[idx_d.at[j - RING]], ssem[bb]).wait()

            @pl.when(j < NCH)
            def _():
                pltpu.async_copy(
                    hp_hbm.at[idx_s.at[pl.ds(pl.multiple_of(j * CH, 8), CH)]],
                    rows[bb], gsem[bb])

            @pl.when((j >= 2) & (j < NCH + 2))
            def _():
                pltpu.make_async_copy(
                    hp_hbm.at[idx_s.at[pl.ds(pl.multiple_of((j - 2) * CH, 8),
                                             CH)]],
                    rows[bs], gsem[bs]).wait()
                pltpu.async_copy(
                    rows[bs], shared.at[idx_d.at[j - 2]], ssem[bs], add=True)
        return carry

    lax.fori_loop(0, (NCH + 2 * RING) // RING + 1, step, 0)
    plsc.subcore_barrier()
    for k in range(nfull):
        pltpu.sync_copy(shared.at[pl.ds(base + k * CH, CH)], rows0)
        pltpu.sync_copy(rows0, out_hbm.at[c, pl.ds(base + k * CH, CH)])
    pltpu.sync_copy(shared.at[pl.ds(base + nfull * CH, tail)],
                    rows0.at[pl.ds(0, tail)])
    pltpu.sync_copy(rows0.at[pl.ds(0, tail)],
                    out_hbm.at[c, pl.ds(base + nfull * CH, tail)])


_agg_call = functools.partial(
    pl.kernel,
    out_type=jax.ShapeDtypeStruct((NC, N, F), jnp.float32),
    mesh=_sc_mesh,
    scratch_types=[
        pltpu.VMEM((EPT,), jnp.int32),
        pltpu.VMEM((NCH, CH), jnp.int32),
        pltpu.VMEM((CH, F), jnp.float32),
        pltpu.VMEM((CH, F), jnp.float32),
        pltpu.VMEM((CH, F), jnp.float32),
        pltpu.SemaphoreType.DMA,
        pltpu.SemaphoreType.DMA,
        pltpu.SemaphoreType.DMA,
        pltpu.SemaphoreType.DMA,
        pltpu.SemaphoreType.DMA,
        pltpu.SemaphoreType.DMA,
        pltpu.VMEM_SHARED((N, F), jnp.float32),
    ],
    compiler_params=pltpu.CompilerParams(use_tc_tiling_on_sc=False),
)(_agg_body)


def _pre_body(x_ref, w_ref, deg_ref, hp_ref, dis_ref):
    d = deg_ref[0, :, 0:1] + deg_ref[1, :, 0:1] + 1.0
    dis = jnp.broadcast_to(lax.rsqrt(d), (BLK, F))
    h = jnp.dot(x_ref[...], w_ref[...], preferred_element_type=jnp.float32)
    hp_ref[...] = dis * h
    dis_ref[...] = dis


def _pre(x, w0, degp):
    return pl.pallas_call(
        _pre_body,
        grid=(NBLK,),
        in_specs=[
            pl.BlockSpec((BLK, F), lambda i: (i, 0)),
            pl.BlockSpec((F, F), lambda i: (0, 0)),
            pl.BlockSpec((NC, BLK, DEGW), lambda i: (0, i, 0)),
        ],
        out_specs=[
            pl.BlockSpec((BLK, F), lambda i: (i, 0)),
            pl.BlockSpec((BLK, F), lambda i: (i, 0)),
        ],
        out_shape=[
            jax.ShapeDtypeStruct((N, F), jnp.float32),
            jax.ShapeDtypeStruct((N, F), jnp.float32),
        ],
    )(x, w0, degp)


def _layer_body(agg_ref, hp_ref, dis_ref, b_ref, g_ref, be_ref, w_ref,
                o_ref, s_scr, st_scr):
    p = pl.program_id(0)
    i = pl.program_id(1)

    @pl.when(p == 0)
    def _():
        t = (dis_ref[...] * (agg_ref[0] + agg_ref[1] + hp_ref[...])
             + b_ref[...])
        sv = jax.nn.sigmoid(t)
        s_scr[pl.ds(i * BLK, BLK), :] = sv

        @pl.when(i == 0)
        def _():
            st_scr[...] = jnp.zeros_like(st_scr)

        st_scr[0:1, :] += jnp.sum(sv, axis=0, keepdims=True)
        st_scr[1:2, :] += jnp.sum(sv * sv, axis=0, keepdims=True)

    @pl.when(p == 1)
    def _():
        mean = st_scr[0:1, :] / N
        var = st_scr[1:2, :] / N - mean * mean
        sv = s_scr[pl.ds(i * BLK, BLK), :]
        y = (sv - mean) * lax.rsqrt(var + EPS_BN) * g_ref[...] + be_ref[...]
        o_ref[...] = dis_ref[...] * jnp.dot(
            y, w_ref[...], preferred_element_type=jnp.float32)


def _layer(aggp, hp, dis, b, g, be, wnext):
    return pl.pallas_call(
        _layer_body,
        grid=(2, NBLK),
        in_specs=[
            pl.BlockSpec((NC, BLK, F), lambda p, i: (0, i, 0)),
            pl.BlockSpec((BLK, F), lambda p, i: (i, 0)),
            pl.BlockSpec((BLK, F), lambda p, i: (i, 0)),
            pl.BlockSpec((1, F), lambda p, i: (0, 0)),
            pl.BlockSpec((1, F), lambda p, i: (0, 0)),
            pl.BlockSpec((1, F), lambda p, i: (0, 0)),
            pl.BlockSpec((F, F), lambda p, i: (0, 0)),
        ],
        out_specs=pl.BlockSpec((BLK, F), lambda p, i: (i, 0)),
        out_shape=jax.ShapeDtypeStruct((N, F), jnp.float32),
        scratch_shapes=[
            pltpu.VMEM((N, F), jnp.float32),
            pltpu.VMEM((8, F), jnp.float32),
        ],
    )(aggp, hp, dis, b, g, be, wnext)


def _last_body(agg_ref, hp_ref, dis_ref, b_ref, g_ref, be_ref, batch_ref,
               h_ref, xp_ref, s_scr, st_scr, ps_scr, pc_scr):
    p = pl.program_id(0)
    i = pl.program_id(1)

    @pl.when(p == 0)
    def _():
        t = (dis_ref[...] * (agg_ref[0] + agg_ref[1] + hp_ref[...])
             + b_ref[...])
        sv = jax.nn.sigmoid(t)
        s_scr[pl.ds(i * BLK, BLK), :] = sv

        @pl.when(i == 0)
        def _():
            st_scr[...] = jnp.zeros_like(st_scr)

        st_scr[0:1, :] += jnp.sum(sv, axis=0, keepdims=True)
        st_scr[1:2, :] += jnp.sum(sv * sv, axis=0, keepdims=True)

    @pl.when(p == 1)
    def _():
        mean = st_scr[0:1, :] / N
        var = st_scr[1:2, :] / N - mean * mean
        sv = s_scr[pl.ds(i * BLK, BLK), :]
        y = (sv - mean) * lax.rsqrt(var + EPS_BN) * g_ref[...] + be_ref[...]
        h_ref[...] = y

        oh = (batch_ref[...] == lax.broadcasted_iota(jnp.int32, (BLK, G), 1)
              ).astype(jnp.float32)

        @pl.when(i == 0)
        def _():
            ps_scr[...] = jnp.zeros_like(ps_scr)
            pc_scr[...] = jnp.zeros_like(pc_scr)

        ps_scr[...] += lax.dot_general(oh, y, (((0,), (0,)), ((), ())),
                                       preferred_element_type=jnp.float32)
        pc_scr[...] += jnp.sum(oh, axis=0, keepdims=True)

        @pl.when(i == NBLK - 1)
        def _():
            xp_ref[...] = (ps_scr[...]
                           / jnp.maximum(pc_scr[...], 1.0).reshape(G, 1))


def _last(aggp, hp, dis, b, g, be, batch2):
    return pl.pallas_call(
        _last_body,
        grid=(2, NBLK),
        in_specs=[
            pl.BlockSpec((NC, BLK, F), lambda p, i: (0, i, 0)),
            pl.BlockSpec((BLK, F), lambda p, i: (i, 0)),
            pl.BlockSpec((BLK, F), lambda p, i: (i, 0)),
            pl.BlockSpec((1, F), lambda p, i: (0, 0)),
            pl.BlockSpec((1, F), lambda p, i: (0, 0)),
            pl.BlockSpec((1, F), lambda p, i: (0, 0)),
            pl.BlockSpec((BLK, 1), lambda p, i: (i, 0)),
        ],
        out_specs=[
            pl.BlockSpec((BLK, F), lambda p, i: (i, 0)),
            pl.BlockSpec((G, G), lambda p, i: (0, 0)),
        ],
        out_shape=[
            jax.ShapeDtypeStruct((N, F), jnp.float32),
            jax.ShapeDtypeStruct((G, G), jnp.float32),
        ],
        scratch_shapes=[
            pltpu.VMEM((N, F), jnp.float32),
            pltpu.VMEM((8, F), jnp.float32),
            pltpu.VMEM((G, G), jnp.float32),
            pltpu.VMEM((1, G), jnp.float32),
        ],
    )(aggp, hp, dis, b, g, be, batch2)


def kernel(x, edge_index, batch, W0, b0, g0, be0, W1, b1, g1, be1,
           W2, b2, g2, be2):
    src = edge_index[0]
    dst = edge_index[1].reshape(TILES, NCH, CH)
    batch2 = batch.reshape(N, 1)
    ones_deg = jnp.ones((CH, DEGW), jnp.float32)
    zeros_deg = jnp.zeros((RSTG, DEGW), jnp.float32)
    zeros_f = jnp.zeros((CH, F), jnp.float32)

    degp = _deg_call(dst, ones_deg, zeros_deg)
    hp, dis = _pre(x, W0, degp)

    params = [(b0, g0, be0, W1), (b1, g1, be1, W2), (b2, g2, be2, None)]
    for (b, g, be, wnext) in params:
        aggp = _agg_call(hp, src, dst, zeros_f)
        if wnext is not None:
            hp = _layer(aggp, hp, dis, b.reshape(1, F), g.reshape(1, F),
                        be.reshape(1, F), wnext)
        else:
            h, xpool = _last(aggp, hp, dis, b.reshape(1, F), g.reshape(1, F),
                             be.reshape(1, F), batch2)
    return (xpool, h)
